# Initial kernel scaffold; baseline (speedup 1.0000x reference)
import jax
import jax.numpy as jnp
from jax.experimental import pallas as pl


def kernel(x, edge_index, edge_weight, W, b):
    raise NotImplementedError("write your pallas kernel here")


# speedup vs baseline: 21.5272x; 21.5272x over previous
import functools

import jax
import jax.numpy as jnp
from jax import lax
from jax.experimental import pallas as pl
from jax.experimental.pallas import tpu as pltpu
from jax.experimental.pallas import tpu_sc as plsc

N = 10000
D = 128
E = 320000
NC = 2
NS = 16
NW = NC * NS
CH = 128
K = -(-E // (NW * CH))
EP = NW * K * CH
K2 = -(-E // (NS * CH))
EP2 = NS * K2 * CH
N_PAD = 10240
SEG = N_PAD // NS
DH = D // NC

_mesh = plsc.VectorSubcoreMesh(core_axis_name="c", subcore_axis_name="s")


@functools.partial(
    pl.kernel,
    out_type=jax.ShapeDtypeStruct((NC, N_PAD), jnp.float32),
    mesh=_mesh,
    scratch_types=[
        pltpu.VMEM((K, CH), jnp.int32),
        pltpu.VMEM((K, CH), jnp.float32),
        pltpu.VMEM((SEG,), jnp.float32),
        pltpu.VMEM_SHARED((N_PAD,), jnp.float32),
    ],
)
def _deg_kernel(row_hbm, w_hbm, deg_out, row_slab, w_slab, zbuf, deg_sh):
    c = lax.axis_index("c")
    s = lax.axis_index("s")
    wid = c * NS + s
    pltpu.sync_copy(row_hbm.at[wid], row_slab)
    pltpu.sync_copy(w_hbm.at[wid], w_slab)

    def _zero(i, carry):
        zbuf[pl.ds(i * 16, 16)] = jnp.zeros((16,), jnp.float32)
        return carry

    lax.fori_loop(0, SEG // 16, _zero, 0)
    pltpu.sync_copy(zbuf, deg_sh.at[pl.ds(s * SEG, SEG)])
    plsc.subcore_barrier()

    def _body(j, carry):
        pltpu.sync_copy(w_slab.at[j], deg_sh.at[row_slab.at[j]], add=True)
        return carry

    lax.fori_loop(0, K, _body, 0)
    plsc.subcore_barrier()
    pltpu.sync_copy(deg_sh.at[pl.ds(s * SEG, SEG)],
                    deg_out.at[c, pl.ds(s * SEG, SEG)])


@functools.partial(
    pl.kernel,
    out_type=jax.ShapeDtypeStruct((NC, N_PAD, DH), jnp.float32),
    mesh=_mesh,
    scratch_types=[
        pltpu.VMEM((K2, CH), jnp.int32),
        pltpu.VMEM((K2, CH), jnp.int32),
        pltpu.VMEM((CH, DH), jnp.float32),
        pltpu.VMEM((CH, DH), jnp.float32),
        pltpu.SemaphoreType.DMA,
        pltpu.VMEM_SHARED((N_PAD, DH), jnp.float32),
    ],
    compiler_params=pltpu.CompilerParams(use_tc_tiling_on_sc=False),
)
def _msg_kernel(y_hbm, col_hbm, row_hbm, acc_out,
                col_slab, row_slab, gbuf, zbuf, sem, acc_sh):
    c = lax.axis_index("c")
    s = lax.axis_index("s")
    pltpu.sync_copy(col_hbm.at[s], col_slab)
    pltpu.sync_copy(row_hbm.at[s], row_slab)

    def _zero(i, carry):
        zbuf[i // (DH // 16), pl.ds((i % (DH // 16)) * 16, 16)] = (
            jnp.zeros((16,), jnp.float32))
        return carry

    lax.fori_loop(0, CH * (DH // 16), _zero, 0)
    for t in range(SEG // CH):
        pltpu.sync_copy(zbuf, acc_sh.at[pl.ds(s * SEG + t * CH, CH)])
    plsc.subcore_barrier()

    def _body(j, carry):
        pltpu.async_copy(y_hbm.at[c].at[col_slab.at[j]], gbuf, sem).wait()
        pltpu.sync_copy(gbuf, acc_sh.at[row_slab.at[j]], add=True)
        return carry

    lax.fori_loop(0, K2, _body, 0)
    plsc.subcore_barrier()
    pltpu.sync_copy(acc_sh.at[pl.ds(s * SEG, SEG)],
                    acc_out.at[c, pl.ds(s * SEG, SEG)])


def _y_body(x_ref, w_ref, deg_ref, y_ref):
    deg = deg_ref[0, :] + deg_ref[1, :]
    dis = jnp.where(deg > 0, lax.rsqrt(deg), 0.0)
    xw = jnp.dot(x_ref[...], w_ref[0], preferred_element_type=jnp.float32)
    y_ref[0] = xw * dis[:, None]


def _out_body(acc_ref, deg_ref, b_ref, o_ref):
    deg = deg_ref[0, :] + deg_ref[1, :]
    dis = jnp.where(deg > 0, lax.rsqrt(deg), 0.0)
    acc = jnp.concatenate([acc_ref[0], acc_ref[1]], axis=-1)
    o_ref[...] = acc * dis[:, None] + b_ref[...]


_RB = 1024


def kernel(x, edge_index, edge_weight, W, b):
    row = edge_index[0].astype(jnp.int32)
    col = edge_index[1].astype(jnp.int32)
    w = edge_weight.astype(jnp.float32)

    pad = EP - E
    pad_rows = N + (jnp.arange(pad, dtype=jnp.int32) % (N_PAD - N))
    row_p = jnp.concatenate([row, pad_rows]).reshape(NW, K, CH)
    w_p = jnp.concatenate([w, jnp.zeros((pad,), jnp.float32)]).reshape(NW, K, CH)

    pad2 = EP2 - E
    pad_rows2 = N + (jnp.arange(pad2, dtype=jnp.int32) % (N_PAD - N))
    pad_cols2 = jnp.arange(pad2, dtype=jnp.int32) % N
    row_p2 = jnp.concatenate([row, pad_rows2]).reshape(NS, K2, CH)
    col_p2 = jnp.concatenate([col, pad_cols2]).reshape(NS, K2, CH)

    x_p = jnp.pad(x, ((0, N_PAD - N), (0, 0)))

    deg2 = _deg_kernel(row_p, w_p)

    y = pl.pallas_call(
        _y_body,
        grid=(N_PAD // _RB, NC),
        in_specs=[
            pl.BlockSpec((_RB, D), lambda i, h: (i, 0)),
            pl.BlockSpec((1, D, DH), lambda i, h: (h, 0, 0)),
            pl.BlockSpec((NC, _RB), lambda i, h: (0, i)),
        ],
        out_specs=pl.BlockSpec((1, _RB, DH), lambda i, h: (h, i, 0)),
        out_shape=jax.ShapeDtypeStruct((NC, N_PAD, DH), jnp.float32),
    )(x_p, W.reshape(D, NC, DH).transpose(1, 0, 2), deg2)

    acc2 = _msg_kernel(y, col_p2, row_p2)

    out_full = pl.pallas_call(
        _out_body,
        grid=(N_PAD // _RB,),
        in_specs=[
            pl.BlockSpec((NC, _RB, DH), lambda i: (0, i, 0)),
            pl.BlockSpec((NC, _RB), lambda i: (0, i)),
            pl.BlockSpec((1, D), lambda i: (0, 0)),
        ],
        out_specs=pl.BlockSpec((_RB, D), lambda i: (i, 0)),
        out_shape=jax.ShapeDtypeStruct((N_PAD, D), jnp.float32),
    )(acc2, deg2, b.reshape(1, D))

    return out_full[:N]


# speedup vs baseline: 33.7063x; 1.5658x over previous
import functools

import jax
import jax.numpy as jnp
from jax import lax
from jax.experimental import pallas as pl
from jax.experimental.pallas import tpu as pltpu
from jax.experimental.pallas import tpu_sc as plsc

N = 10000
D = 128
E = 320000
NC = 2
NS = 16
NW = NC * NS
CH = 128
K = -(-E // (NW * CH))
EP = NW * K * CH
NBUF = 4
LEAD = 2
K2 = -(-E // (NS * CH * NBUF)) * NBUF
EP2 = NS * K2 * CH
N_PAD = 10240
SEG = N_PAD // NS
DH = D // NC

_mesh = plsc.VectorSubcoreMesh(core_axis_name="c", subcore_axis_name="s")


@functools.partial(
    pl.kernel,
    out_type=jax.ShapeDtypeStruct((NC, N_PAD), jnp.float32),
    mesh=_mesh,
    scratch_types=[
        pltpu.VMEM((K, CH), jnp.int32),
        pltpu.VMEM((K, CH), jnp.float32),
        pltpu.VMEM((SEG,), jnp.float32),
        pltpu.VMEM_SHARED((N_PAD,), jnp.float32),
    ],
)
def _deg_kernel(row_hbm, w_hbm, deg_out, row_slab, w_slab, zbuf, deg_sh):
    c = lax.axis_index("c")
    s = lax.axis_index("s")
    wid = c * NS + s
    pltpu.sync_copy(row_hbm.at[wid], row_slab)
    pltpu.sync_copy(w_hbm.at[wid], w_slab)

    def _zero(i, carry):
        zbuf[pl.ds(i * 16, 16)] = jnp.zeros((16,), jnp.float32)
        return carry

    lax.fori_loop(0, SEG // 16, _zero, 0)
    pltpu.sync_copy(zbuf, deg_sh.at[pl.ds(s * SEG, SEG)])
    plsc.subcore_barrier()

    def _body(j, carry):
        pltpu.sync_copy(w_slab.at[j], deg_sh.at[row_slab.at[j]], add=True)
        return carry

    lax.fori_loop(0, K, _body, 0)
    plsc.subcore_barrier()
    pltpu.sync_copy(deg_sh.at[pl.ds(s * SEG, SEG)],
                    deg_out.at[c, pl.ds(s * SEG, SEG)])


@functools.partial(
    pl.kernel,
    out_type=jax.ShapeDtypeStruct((NC, N_PAD, DH), jnp.float32),
    mesh=_mesh,
    scratch_types=[
        pltpu.VMEM((K2, CH), jnp.int32),
        pltpu.VMEM((K2, CH), jnp.int32),
        pltpu.VMEM((NBUF, CH, DH), jnp.float32),
        pltpu.VMEM((CH, DH), jnp.float32),
        pltpu.SemaphoreType.DMA((NBUF,)),
        pltpu.SemaphoreType.DMA((NBUF,)),
        pltpu.VMEM_SHARED((N_PAD, DH), jnp.float32),
    ],
    compiler_params=pltpu.CompilerParams(use_tc_tiling_on_sc=False),
)
def _msg_kernel(y_hbm, col_hbm, row_hbm, acc_out,
                col_slab, row_slab, gbuf, zbuf, gsem, ssem, acc_sh):
    c = lax.axis_index("c")
    s = lax.axis_index("s")
    pltpu.sync_copy(col_hbm.at[s], col_slab)
    pltpu.sync_copy(row_hbm.at[s], row_slab)

    def _zero(i, carry):
        zbuf[i // (DH // 16), pl.ds((i % (DH // 16)) * 16, 16)] = (
            jnp.zeros((16,), jnp.float32))
        return carry

    lax.fori_loop(0, CH * (DH // 16), _zero, 0)
    for t in range(SEG // CH):
        pltpu.sync_copy(zbuf, acc_sh.at[pl.ds(s * SEG + t * CH, CH)])
    plsc.subcore_barrier()

    def _start_gather(j, b):
        pltpu.async_copy(y_hbm.at[c].at[col_slab.at[j]], gbuf.at[b],
                         gsem.at[b])

    def _wait_gather(b):
        pltpu.make_async_copy(y_hbm.at[c].at[col_slab.at[0]], gbuf.at[b],
                              gsem.at[b]).wait()

    def _start_scatter(j, b):
        pltpu.async_copy(gbuf.at[b], acc_sh.at[row_slab.at[j]], ssem.at[b],
                         add=True)

    def _wait_scatter(b):
        pltpu.make_async_copy(gbuf.at[b], acc_sh.at[row_slab.at[0]],
                              ssem.at[b]).wait()

    for b in range(LEAD):
        _start_gather(b, b)

    def _group(g, carry):
        for b in range(NBUF):
            j = g * NBUF + b
            b2 = (b + LEAD) % NBUF
            _wait_gather(b)
            _start_scatter(j, b)
            jn = j + LEAD

            @pl.when(jnp.logical_and(j >= NBUF - LEAD, jn < K2))
            def _():
                _wait_scatter(b2)

            @pl.when(jn < K2)
            def _():
                _start_gather(jn, b2)
        return carry

    lax.fori_loop(0, K2 // NBUF, _group, 0)
    for b in range(NBUF):
        _wait_scatter(b)
    plsc.subcore_barrier()
    pltpu.sync_copy(acc_sh.at[pl.ds(s * SEG, SEG)],
                    acc_out.at[c, pl.ds(s * SEG, SEG)])


def _y_body(x_ref, w_ref, deg_ref, y_ref):
    deg = deg_ref[0, :] + deg_ref[1, :]
    dis = jnp.where(deg > 0, lax.rsqrt(deg), 0.0)
    xw = jnp.dot(x_ref[...], w_ref[0], preferred_element_type=jnp.float32)
    y_ref[0] = xw * dis[:, None]


def _out_body(acc_ref, deg_ref, b_ref, o_ref):
    deg = deg_ref[0, :] + deg_ref[1, :]
    dis = jnp.where(deg > 0, lax.rsqrt(deg), 0.0)
    acc = jnp.concatenate([acc_ref[0], acc_ref[1]], axis=-1)
    o_ref[...] = acc * dis[:, None] + b_ref[...]


_RB = 2048


def kernel(x, edge_index, edge_weight, W, b):
    row = edge_index[0].astype(jnp.int32)
    col = edge_index[1].astype(jnp.int32)
    w = edge_weight.astype(jnp.float32)

    pad = EP - E
    pad_rows = N + (jnp.arange(pad, dtype=jnp.int32) % (N_PAD - N))
    row_p = jnp.concatenate([row, pad_rows]).reshape(NW, K, CH)
    w_p = jnp.concatenate([w, jnp.zeros((pad,), jnp.float32)]).reshape(NW, K, CH)

    pad2 = EP2 - E
    pad_rows2 = N + (jnp.arange(pad2, dtype=jnp.int32) % (N_PAD - N))
    pad_cols2 = jnp.arange(pad2, dtype=jnp.int32) % N
    row_p2 = jnp.concatenate([row, pad_rows2]).reshape(NS, K2, CH)
    col_p2 = jnp.concatenate([col, pad_cols2]).reshape(NS, K2, CH)

    deg2 = _deg_kernel(row_p, w_p)

    y = pl.pallas_call(
        _y_body,
        grid=(-(-N // _RB), NC),
        in_specs=[
            pl.BlockSpec((_RB, D), lambda i, h: (i, 0)),
            pl.BlockSpec((1, D, DH), lambda i, h: (h, 0, 0)),
            pl.BlockSpec((NC, _RB), lambda i, h: (0, i)),
        ],
        out_specs=pl.BlockSpec((1, _RB, DH), lambda i, h: (h, i, 0)),
        out_shape=jax.ShapeDtypeStruct((NC, N, DH), jnp.float32),
    )(x, W.reshape(D, NC, DH).transpose(1, 0, 2), deg2)

    acc2 = _msg_kernel(y, col_p2, row_p2)

    out = pl.pallas_call(
        _out_body,
        grid=(-(-N // _RB),),
        in_specs=[
            pl.BlockSpec((NC, _RB, DH), lambda i: (0, i, 0)),
            pl.BlockSpec((NC, _RB), lambda i: (0, i)),
            pl.BlockSpec((1, D), lambda i: (0, 0)),
        ],
        out_specs=pl.BlockSpec((_RB, D), lambda i: (i, 0)),
        out_shape=jax.ShapeDtypeStruct((N, D), jnp.float32),
    )(acc2, deg2, b.reshape(1, D))

    return out


# speedup vs baseline: 37.3350x; 1.1077x over previous
import functools

import jax
import jax.numpy as jnp
from jax import lax
from jax.experimental import pallas as pl
from jax.experimental.pallas import tpu as pltpu
from jax.experimental.pallas import tpu_sc as plsc

N = 10000
D = 128
E = 320000
NC = 2
NS = 16
NW = NC * NS
CH = 128
K = -(-E // (NW * CH))
EP = NW * K * CH
NBUF = 5
LEAD = 3
K2 = -(-E // (NS * CH * NBUF)) * NBUF
EP2 = NS * K2 * CH
N_PAD = 10240
SEG = N_PAD // NS
DH = D // NC

_mesh = plsc.VectorSubcoreMesh(core_axis_name="c", subcore_axis_name="s")


@functools.partial(
    pl.kernel,
    out_type=jax.ShapeDtypeStruct((NC, N_PAD), jnp.float32),
    mesh=_mesh,
    scratch_types=[
        pltpu.VMEM((K, CH), jnp.int32),
        pltpu.VMEM((K, CH), jnp.float32),
        pltpu.VMEM((SEG,), jnp.float32),
        pltpu.VMEM_SHARED((N_PAD,), jnp.float32),
    ],
)
def _deg_kernel(row_hbm, w_hbm, deg_out, row_slab, w_slab, zbuf, deg_sh):
    c = lax.axis_index("c")
    s = lax.axis_index("s")
    wid = c * NS + s
    pltpu.sync_copy(row_hbm.at[wid], row_slab)
    pltpu.sync_copy(w_hbm.at[wid], w_slab)

    def _zero(i, carry):
        zbuf[pl.ds(i * 16, 16)] = jnp.zeros((16,), jnp.float32)
        return carry

    lax.fori_loop(0, SEG // 16, _zero, 0)
    pltpu.sync_copy(zbuf, deg_sh.at[pl.ds(s * SEG, SEG)])
    plsc.subcore_barrier()

    def _body(j, carry):
        pltpu.sync_copy(w_slab.at[j], deg_sh.at[row_slab.at[j]], add=True)
        return carry

    lax.fori_loop(0, K, _body, 0)
    plsc.subcore_barrier()
    pltpu.sync_copy(deg_sh.at[pl.ds(s * SEG, SEG)],
                    deg_out.at[c, pl.ds(s * SEG, SEG)])


@functools.partial(
    pl.kernel,
    out_type=jax.ShapeDtypeStruct((NC, N_PAD, DH), jnp.float32),
    mesh=_mesh,
    scratch_types=[
        pltpu.VMEM((K2, CH), jnp.int32),
        pltpu.VMEM((K2, CH), jnp.int32),
        pltpu.VMEM((NBUF, CH, DH), jnp.float32),
        pltpu.VMEM((CH, DH), jnp.float32),
        pltpu.SemaphoreType.DMA((NBUF,)),
        pltpu.SemaphoreType.DMA((NBUF,)),
        pltpu.VMEM_SHARED((N_PAD, DH), jnp.float32),
    ],
    compiler_params=pltpu.CompilerParams(use_tc_tiling_on_sc=False),
)
def _msg_kernel(y_hbm, col_hbm, row_hbm, acc_out,
                col_slab, row_slab, gbuf, zbuf, gsem, ssem, acc_sh):
    c = lax.axis_index("c")
    s = lax.axis_index("s")
    pltpu.sync_copy(col_hbm.at[s], col_slab)
    pltpu.sync_copy(row_hbm.at[s], row_slab)

    def _zero(i, carry):
        zbuf[i // (DH // 16), pl.ds((i % (DH // 16)) * 16, 16)] = (
            jnp.zeros((16,), jnp.float32))
        return carry

    lax.fori_loop(0, CH * (DH // 16), _zero, 0)
    for t in range(SEG // CH):
        pltpu.sync_copy(zbuf, acc_sh.at[pl.ds(s * SEG + t * CH, CH)])
    plsc.subcore_barrier()

    def _start_gather(j, b):
        pltpu.async_copy(y_hbm.at[c].at[col_slab.at[j]], gbuf.at[b],
                         gsem.at[b])

    def _wait_gather(b):
        pltpu.make_async_copy(y_hbm.at[c].at[col_slab.at[0]], gbuf.at[b],
                              gsem.at[b]).wait()

    def _start_scatter(j, b):
        pltpu.async_copy(gbuf.at[b], acc_sh.at[row_slab.at[j]], ssem.at[b],
                         add=True)

    def _wait_scatter(b):
        pltpu.make_async_copy(gbuf.at[b], acc_sh.at[row_slab.at[0]],
                              ssem.at[b]).wait()

    for b in range(LEAD):
        _start_gather(b, b)

    def _group(g, carry):
        for b in range(NBUF):
            j = g * NBUF + b
            b2 = (b + LEAD) % NBUF
            _wait_gather(b)
            _start_scatter(j, b)
            jn = j + LEAD

            @pl.when(jnp.logical_and(j >= NBUF - LEAD, jn < K2))
            def _():
                _wait_scatter(b2)

            @pl.when(jn < K2)
            def _():
                _start_gather(jn, b2)
        return carry

    lax.fori_loop(0, K2 // NBUF, _group, 0)
    for b in range(NBUF):
        _wait_scatter(b)
    plsc.subcore_barrier()
    pltpu.sync_copy(acc_sh.at[pl.ds(s * SEG, SEG)],
                    acc_out.at[c, pl.ds(s * SEG, SEG)])


def _y_body(x_ref, w_ref, deg_ref, y_ref):
    deg = deg_ref[0, :] + deg_ref[1, :]
    dis = jnp.where(deg > 0, lax.rsqrt(deg), 0.0)
    xw = jnp.dot(x_ref[...], w_ref[0], preferred_element_type=jnp.float32)
    y_ref[0] = xw * dis[:, None]


def _out_body(acc_ref, deg_ref, b_ref, o_ref):
    deg = deg_ref[0, :] + deg_ref[1, :]
    dis = jnp.where(deg > 0, lax.rsqrt(deg), 0.0)
    acc = jnp.concatenate([acc_ref[0], acc_ref[1]], axis=-1)
    o_ref[...] = acc * dis[:, None] + b_ref[...]


_RB = 2048


def kernel(x, edge_index, edge_weight, W, b):
    row = edge_index[0].astype(jnp.int32)
    col = edge_index[1].astype(jnp.int32)
    w = edge_weight.astype(jnp.float32)

    pad = EP - E
    pad_rows = N + (jnp.arange(pad, dtype=jnp.int32) % (N_PAD - N))
    row_p = jnp.concatenate([row, pad_rows]).reshape(NW, K, CH)
    w_p = jnp.concatenate([w, jnp.zeros((pad,), jnp.float32)]).reshape(NW, K, CH)

    pad2 = EP2 - E
    pad_rows2 = N + (jnp.arange(pad2, dtype=jnp.int32) % (N_PAD - N))
    pad_cols2 = jnp.arange(pad2, dtype=jnp.int32) % N
    row_p2 = jnp.concatenate([row, pad_rows2]).reshape(NS, K2, CH)
    col_p2 = jnp.concatenate([col, pad_cols2]).reshape(NS, K2, CH)

    deg2 = _deg_kernel(row_p, w_p)

    y = pl.pallas_call(
        _y_body,
        grid=(-(-N // _RB), NC),
        in_specs=[
            pl.BlockSpec((_RB, D), lambda i, h: (i, 0)),
            pl.BlockSpec((1, D, DH), lambda i, h: (h, 0, 0)),
            pl.BlockSpec((NC, _RB), lambda i, h: (0, i)),
        ],
        out_specs=pl.BlockSpec((1, _RB, DH), lambda i, h: (h, i, 0)),
        out_shape=jax.ShapeDtypeStruct((NC, N, DH), jnp.float32),
    )(x, W.reshape(D, NC, DH).transpose(1, 0, 2), deg2)

    acc2 = _msg_kernel(y, col_p2, row_p2)

    out = pl.pallas_call(
        _out_body,
        grid=(-(-N // _RB),),
        in_specs=[
            pl.BlockSpec((NC, _RB, DH), lambda i: (0, i, 0)),
            pl.BlockSpec((NC, _RB), lambda i: (0, i)),
            pl.BlockSpec((1, D), lambda i: (0, 0)),
        ],
        out_specs=pl.BlockSpec((_RB, D), lambda i: (i, 0)),
        out_shape=jax.ShapeDtypeStruct((N, D), jnp.float32),
    )(acc2, deg2, b.reshape(1, D))

    return out


# speedup vs baseline: 37.5801x; 1.0066x over previous
import functools

import jax
import jax.numpy as jnp
from jax import lax
from jax.experimental import pallas as pl
from jax.experimental.pallas import tpu as pltpu
from jax.experimental.pallas import tpu_sc as plsc

N = 10000
D = 128
E = 320000
NC = 2
NS = 16
NW = NC * NS
CH = 128
K = -(-E // (NW * CH))
EP = NW * K * CH
NBUF = 5
LEAD = 3
K2 = -(-E // (NS * CH * NBUF)) * NBUF
EP2 = NS * K2 * CH
N_PAD = 10240
SEG = N_PAD // NS
DH = D // NC

_mesh = plsc.VectorSubcoreMesh(core_axis_name="c", subcore_axis_name="s")


@functools.partial(
    pl.kernel,
    out_type=jax.ShapeDtypeStruct((NC, N_PAD), jnp.float32),
    mesh=_mesh,
    scratch_types=[
        pltpu.VMEM((K, CH), jnp.int32),
        pltpu.VMEM((K, CH), jnp.float32),
        pltpu.VMEM((SEG,), jnp.float32),
        pltpu.VMEM_SHARED((N_PAD,), jnp.float32),
    ],
)
def _deg_kernel(row_hbm, w_hbm, deg_out, row_slab, w_slab, zbuf, deg_sh):
    c = lax.axis_index("c")
    s = lax.axis_index("s")
    wid = c * NS + s
    pltpu.sync_copy(row_hbm.at[wid], row_slab)
    pltpu.sync_copy(w_hbm.at[wid], w_slab)

    def _zero(i, carry):
        zbuf[pl.ds(i * 16, 16)] = jnp.zeros((16,), jnp.float32)
        return carry

    lax.fori_loop(0, SEG // 16, _zero, 0)
    pltpu.sync_copy(zbuf, deg_sh.at[pl.ds(s * SEG, SEG)])
    plsc.subcore_barrier()

    def _body(j, carry):
        pltpu.sync_copy(w_slab.at[j], deg_sh.at[row_slab.at[j]], add=True)
        return carry

    lax.fori_loop(0, K, _body, 0)
    plsc.subcore_barrier()
    pltpu.sync_copy(deg_sh.at[pl.ds(s * SEG, SEG)],
                    deg_out.at[c, pl.ds(s * SEG, SEG)])


@functools.partial(
    pl.kernel,
    out_type=jax.ShapeDtypeStruct((N, D), jnp.float32),
    mesh=_mesh,
    scratch_types=[
        pltpu.VMEM((K2, CH), jnp.int32),
        pltpu.VMEM((K2, CH), jnp.int32),
        pltpu.VMEM((NBUF, CH, DH), jnp.float32),
        pltpu.VMEM((SEG,), jnp.float32),
        pltpu.VMEM((DH,), jnp.float32),
        pltpu.SemaphoreType.DMA((NBUF,)),
        pltpu.SemaphoreType.DMA((NBUF,)),
        pltpu.VMEM_SHARED((N_PAD, DH), jnp.float32),
    ],
    compiler_params=pltpu.CompilerParams(use_tc_tiling_on_sc=False),
)
def _msg_kernel(y_hbm, col_hbm, row_hbm, dis_hbm, b_hbm, out_hbm,
                col_slab, row_slab, gbuf, dis_t, bias_t,
                gsem, ssem, acc_sh):
    c = lax.axis_index("c")
    s = lax.axis_index("s")
    pltpu.sync_copy(col_hbm.at[s], col_slab)
    pltpu.sync_copy(row_hbm.at[s], row_slab)
    pltpu.sync_copy(dis_hbm.at[0, pl.ds(s * SEG, SEG)], dis_t)
    pltpu.sync_copy(b_hbm.at[pl.ds(c * DH, DH)], bias_t)

    def _zero(i, carry):
        gbuf[0, i // (DH // 16), pl.ds((i % (DH // 16)) * 16, 16)] = (
            jnp.zeros((16,), jnp.float32))
        return carry

    lax.fori_loop(0, CH * (DH // 16), _zero, 0)
    for t in range(SEG // CH):
        pltpu.sync_copy(gbuf.at[0], acc_sh.at[pl.ds(s * SEG + t * CH, CH)])
    plsc.subcore_barrier()

    def _start_gather(j, b):
        pltpu.async_copy(y_hbm.at[c].at[col_slab.at[j]], gbuf.at[b],
                         gsem.at[b])

    def _wait_gather(b):
        pltpu.make_async_copy(y_hbm.at[c].at[col_slab.at[0]], gbuf.at[b],
                              gsem.at[b]).wait()

    def _start_scatter(j, b):
        pltpu.async_copy(gbuf.at[b], acc_sh.at[row_slab.at[j]], ssem.at[b],
                         add=True)

    def _wait_scatter(b):
        pltpu.make_async_copy(gbuf.at[b], acc_sh.at[row_slab.at[0]],
                              ssem.at[b]).wait()

    for b in range(LEAD):
        _start_gather(b, b)

    def _group(g, carry):
        for b in range(NBUF):
            j = g * NBUF + b
            b2 = (b + LEAD) % NBUF
            _wait_gather(b)
            _start_scatter(j, b)
            jn = j + LEAD

            @pl.when(jnp.logical_and(j >= NBUF - LEAD, jn < K2))
            def _():
                _wait_scatter(b2)

            @pl.when(jn < K2)
            def _():
                _start_gather(jn, b2)
        return carry

    lax.fori_loop(0, K2 // NBUF, _group, 0)
    for b in range(NBUF):
        _wait_scatter(b)
    plsc.subcore_barrier()

    def _scale_chunk(t):
        pltpu.sync_copy(acc_sh.at[pl.ds(s * SEG + t * CH, CH)], gbuf.at[0])

        def _rowgrp(rg, carry):
            dvals = dis_t[pl.ds(t * CH + rg * 16, 16)]
            for i in range(16):
                dval = dvals[i]
                r = rg * 16 + i
                for q in range(DH // 16):
                    sl = pl.ds(q * 16, 16)
                    gbuf[0, r, sl] = gbuf[0, r, sl] * dval + bias_t[sl]
            return carry

        lax.fori_loop(0, CH // 16, _rowgrp, 0)

    for t in range(SEG // CH):
        @pl.when(s * SEG + (t + 1) * CH <= N)
        def _():
            _scale_chunk(t)
            pltpu.sync_copy(gbuf.at[0],
                            out_hbm.at[pl.ds(s * SEG + t * CH, CH),
                                       pl.ds(c * DH, DH)])

    _NFULL = (N - (NS - 1) * SEG) // CH
    _PART = N - (NS - 1) * SEG - _NFULL * CH

    @pl.when(s == NS - 1)
    def _():
        _scale_chunk(_NFULL)
        pltpu.sync_copy(gbuf.at[0].at[pl.ds(0, _PART)],
                        out_hbm.at[pl.ds((NS - 1) * SEG + _NFULL * CH, _PART),
                                   pl.ds(c * DH, DH)])


def _y_body(x_ref, w_ref, deg_ref, y_ref, dis_ref):
    deg = deg_ref[0, :] + deg_ref[1, :]
    dis = jnp.where(deg > 0, lax.rsqrt(deg), 0.0)
    xw = jnp.dot(x_ref[...], w_ref[0], preferred_element_type=jnp.float32)
    y_ref[0] = xw * dis[:, None]
    dis_ref[0, :] = dis
    dis_ref[1, :] = dis


_RB = 2048


def kernel(x, edge_index, edge_weight, W, b):
    row = edge_index[0].astype(jnp.int32)
    col = edge_index[1].astype(jnp.int32)
    w = edge_weight.astype(jnp.float32)

    pad = EP - E
    pad_rows = N + (jnp.arange(pad, dtype=jnp.int32) % (N_PAD - N))
    row_p = jnp.concatenate([row, pad_rows]).reshape(NW, K, CH)
    w_p = jnp.concatenate([w, jnp.zeros((pad,), jnp.float32)]).reshape(NW, K, CH)

    pad2 = EP2 - E
    pad_rows2 = N + (jnp.arange(pad2, dtype=jnp.int32) % (N_PAD - N))
    pad_cols2 = jnp.arange(pad2, dtype=jnp.int32) % N
    row_p2 = jnp.concatenate([row, pad_rows2]).reshape(NS, K2, CH)
    col_p2 = jnp.concatenate([col, pad_cols2]).reshape(NS, K2, CH)

    deg2 = _deg_kernel(row_p, w_p)

    y, dis = pl.pallas_call(
        _y_body,
        grid=(-(-N // _RB), NC),
        in_specs=[
            pl.BlockSpec((_RB, D), lambda i, h: (i, 0)),
            pl.BlockSpec((1, D, DH), lambda i, h: (h, 0, 0)),
            pl.BlockSpec((NC, _RB), lambda i, h: (0, i)),
        ],
        out_specs=[
            pl.BlockSpec((1, _RB, DH), lambda i, h: (h, i, 0)),
            pl.BlockSpec((NC, _RB), lambda i, h: (0, i)),
        ],
        out_shape=[
            jax.ShapeDtypeStruct((NC, N, DH), jnp.float32),
            jax.ShapeDtypeStruct((NC, N_PAD), jnp.float32),
        ],
    )(x, W.reshape(D, NC, DH).transpose(1, 0, 2), deg2)

    return _msg_kernel(y, col_p2, row_p2, dis, b)
